# baseline (speedup 1.0000x reference)
import jax
import jax.numpy as jnp
from jax import lax
from jax.experimental import pallas as pl
from jax.experimental.pallas import tpu as pltpu
from jax.experimental.pallas import tpu_sc as plsc

_N = 10000
_D = 128
_E = 320000
_NC = 2
_NS = 16
_CHUNK = 128
_EC = _E // _NC
_NCHUNK = _EC // _CHUNK
_KFULL = _NCHUNK // _NS
_REM = _NCHUNK - _KFULL * _NS
_SB = 624
_ZR = 16

_NU = 5000
_LANES = _D // 16


def _spmm_body(x0, x1, x2, s0, d0, v0, s1, d1, v1, s2, d2, v2, out,
               idx_s, idx_d, vbuf, rows, acc,
               g0, g1, g2, c0, c1, c2, p0, p1, p2):
    cid = lax.axis_index("c")
    sid = lax.axis_index("s")
    strip = sid * _SB
    tail_base = _NS * _SB
    gsem = (g0, g1, g2)
    csem = (c0, c1, c2)
    psem = (p0, p1, p2)

    z16 = jnp.zeros((16,), jnp.float32)
    xs = (x0, x1, x2)
    srcs = (s0, s1, s2)
    dsts = (d0, d1, d2)
    vals = (v0, v1, v2)

    for rel in range(3):
        xr, sr, dr, vr = xs[rel], srcs[rel], dsts[rel], vals[rel]

        for r in range(_ZR):
            for j in range(_LANES):
                rows[0, r, pl.ds(j * 16, 16)] = z16

        def zero_body(t, _):
            pltpu.async_copy(rows.at[0, pl.ds(0, _ZR)],
                             acc.at[pl.ds(strip + t * _ZR, _ZR)], g0)
            return 0
        lax.fori_loop(0, _SB // _ZR, zero_body, 0)

        @pl.when(sid == _NS - 1)
        def _():
            pltpu.async_copy(rows.at[0, pl.ds(0, _ZR)],
                             acc.at[pl.ds(tail_base, _ZR)], g0)

        def zero_wait(t, _):
            pltpu.make_async_copy(rows.at[0, pl.ds(0, _ZR)],
                                  acc.at[pl.ds(strip, _ZR)], g0).wait()
            return 0
        lax.fori_loop(0, _SB // _ZR, zero_wait, 0)

        @pl.when(sid == _NS - 1)
        def _():
            pltpu.make_async_copy(rows.at[0, pl.ds(0, _ZR)],
                                  acc.at[pl.ds(tail_base, _ZR)], g0).wait()
        plsc.subcore_barrier()

        def ebase(k):
            return cid * _EC + (k * _NS + sid) * _CHUNK

        def start_gather(b):
            pltpu.async_copy(xr.at[idx_s.at[b]], rows.at[b], gsem[b])

        def wait_gather(b):
            pltpu.make_async_copy(xr.at[idx_s.at[b]], rows.at[b],
                                  gsem[b]).wait()

        def start_pf(b, k):
            pltpu.async_copy(sr.at[pl.ds(ebase(k), _CHUNK)], idx_s.at[b],
                             psem[b])
            pltpu.async_copy(vr.at[pl.ds(ebase(k), _CHUNK)], vbuf.at[b],
                             psem[b])
            pltpu.async_copy(dr.at[pl.ds(ebase(k), _CHUNK)], idx_d.at[b],
                             psem[b])

        def wait_pf(b):
            pltpu.make_async_copy(sr.at[pl.ds(0, _CHUNK)], idx_s.at[b],
                                  psem[b]).wait()
            pltpu.make_async_copy(vr.at[pl.ds(0, _CHUNK)], vbuf.at[b],
                                  psem[b]).wait()
            pltpu.make_async_copy(dr.at[pl.ds(0, _CHUNK)], idx_d.at[b],
                                  psem[b]).wait()

        def mult(b):
            @plsc.parallel_loop(0, _CHUNK, 1, unroll=8)
            def _mul(e):
                vb = plsc.load_gather(vbuf.at[b],
                                      [jnp.full((16,), e, jnp.int32)])
                for j in range(_LANES):
                    sl = pl.ds(j * 16, 16)
                    rows[b, e, sl] = rows[b, e, sl] * vb

        def start_scatter(b):
            pltpu.async_copy(rows.at[b], acc.at[idx_d.at[b]], csem[b],
                             add=True)

        def wait_scatter(b):
            pltpu.make_async_copy(xr.at[pl.ds(0, _CHUNK)], rows.at[b],
                                  csem[b]).wait()

        start_pf(0, 0)
        wait_pf(0)
        start_gather(0)
        start_pf(1, 1)
        wait_gather(0)
        wait_pf(1)
        start_gather(1)
        start_pf(2, 2)
        mult(0)
        start_scatter(0)
        wait_gather(1)
        wait_pf(2)
        start_gather(2)
        mult(1)
        start_scatter(1)

        def tri_body(jj, _):
            for i in (0, 1, 2):
                k = 3 * jj + 2 + i
                b = (2 + i) % 3
                o = i
                wait_scatter(o)
                start_pf(o, k + 1)
                wait_gather(b)
                wait_pf(o)
                start_gather(o)
                mult(b)
                start_scatter(b)
            return 0
        lax.fori_loop(0, (_KFULL - 3) // 3, tri_body, 0)
        _lb = (_KFULL - 1) % 3
        wait_gather(_lb)
        mult(_lb)
        start_scatter(_lb)
        wait_scatter((_KFULL - 3) % 3)
        wait_scatter((_KFULL - 2) % 3)
        wait_scatter(_lb)

        if _REM:
            @pl.when(sid < _REM)
            def _():
                base = cid * _EC + (_KFULL * _NS + sid) * _CHUNK
                pltpu.sync_copy(sr.at[pl.ds(base, _CHUNK)], idx_s.at[0])
                pltpu.sync_copy(dr.at[pl.ds(base, _CHUNK)], idx_d.at[0])
                pltpu.sync_copy(vr.at[pl.ds(base, _CHUNK)], vbuf.at[0])
                pltpu.async_copy(xr.at[idx_s.at[0]], rows.at[0], g0).wait()
                mult(0)
                pltpu.sync_copy(rows.at[0], acc.at[idx_d.at[0]], add=True)

        plsc.subcore_barrier()
        pltpu.sync_copy(acc.at[pl.ds(strip, _SB)],
                        out.at[rel, cid, pl.ds(strip, _SB)])

        @pl.when(sid == _NS - 1)
        def _():
            pltpu.sync_copy(acc.at[pl.ds(tail_base, _N - _NS * _SB)],
                            out.at[rel, cid, pl.ds(tail_base,
                                                   _N - _NS * _SB)])
        plsc.subcore_barrier()


_spmm3_cache = []


def _spmm3(*args):
    if not _spmm3_cache:
        _spmm3_cache.append(pl.kernel(
            _spmm_body,
            out_type=jax.ShapeDtypeStruct((3, _NC, _N, _D), jnp.float32),
            mesh=plsc.VectorSubcoreMesh(core_axis_name="c",
                                        subcore_axis_name="s",
                                        num_cores=_NC, num_subcores=_NS),
            compiler_params=pltpu.CompilerParams(needs_layout_passes=False),
            scratch_types=[
                pltpu.VMEM((3, _CHUNK), jnp.int32),
                pltpu.VMEM((3, _CHUNK), jnp.int32),
                pltpu.VMEM((3, _CHUNK), jnp.float32),
                pltpu.VMEM((3, _CHUNK, _D), jnp.float32),
                pltpu.VMEM_SHARED((_N, _D), jnp.float32),
            ] + [pltpu.SemaphoreType.DMA] * 9,
        ))
    return _spmm3_cache[0](*args)


_B = 1000


def _gc_es(p_ref, r_ref, w_ref):
    es = []
    r = r_ref[...]
    w = w_ref[...]
    for i in range(3):
        e = p_ref[i, 0] + p_ref[i, 1]
        e = e * r[i:i + 1, :]
        e = jnp.dot(e, w, preferred_element_type=jnp.float32)
        e = jnp.where(e >= 0.0, e, 0.01 * e)
        es.append(e)
    return es


def _attend(es, scale):
    s = {}
    for a in range(3):
        for b in range(a, 3):
            s[(a, b)] = jnp.sum(es[a] * es[b], axis=1, keepdims=True) * scale
            s[(b, a)] = s[(a, b)]
    outs = []
    for a in range(3):
        m = jnp.maximum(jnp.maximum(s[(a, 0)], s[(a, 1)]), s[(a, 2)])
        w = [jnp.exp(s[(a, b)] - m) for b in range(3)]
        den = w[0] + w[1] + w[2]
        outs.append((w[0] * es[0] + w[1] * es[1] + w[2] * es[2]) / den)
    return outs


def _tc1_body(p_ref, r_ref, w_ref, o0, o1, o2):
    es = _gc_es(p_ref, r_ref, w_ref)
    egos = _attend(es, 128.0 ** -0.5)
    o0[...] = egos[0]
    o1[...] = egos[1]
    o2[...] = egos[2]


def _tc2_body(p_ref, r_ref, w_ref, base_ref, e10, e11, e12, out_ref):
    es = _gc_es(p_ref, r_ref, w_ref)
    egos = _attend(es, 128.0 ** -0.5)
    b = base_ref[...]
    alls = [b + (e10, e11, e12)[i][...] + egos[i] for i in range(3)]
    t = [jnp.sum(alls[2] * alls[j], axis=1, keepdims=True) for j in range(3)]
    m = jnp.maximum(jnp.maximum(t[0], t[1]), t[2])
    w = [jnp.exp(t[j] - m) for j in range(3)]
    den = w[0] + w[1] + w[2]
    mid2 = (w[0] * alls[0] + w[1] * alls[1] + w[2] * alls[2]) / den
    out_ref[:, 0, :] = alls[0] / 3.0
    out_ref[:, 1, :] = alls[1] / 3.0
    out_ref[:, 2, :] = mid2 / 3.0


def _rela_body(r0_ref, w0_ref, w1_ref, r1_out, rm_out):
    r0 = r0_ref[...]
    r1 = jnp.dot(r0, w0_ref[...], preferred_element_type=jnp.float32)
    r2 = jnp.dot(r1, w1_ref[...], preferred_element_type=jnp.float32)
    r1_out[...] = r1
    rm_out[...] = (r0 + r1 + r2) / 3.0


def _gru_body(f_ref, g0, g1, g2, gb_ref, tra_ref, s1_ref, s2_ref):
    u0 = f_ref[:, 0, :]
    u1 = f_ref[:, 1, :]
    u2 = f_ref[:, 2, :]
    gb = gb_ref[...]
    tra = tra_ref[...]
    a1 = u0 * (jnp.dot(u0, g0[...], preferred_element_type=jnp.float32) + gb[0:1, :])
    a2 = u1 * (jnp.dot(u1, g1[...], preferred_element_type=jnp.float32) + gb[1:2, :])
    tg = u2 * (jnp.dot(u2, g2[...], preferred_element_type=jnp.float32) + gb[2:3, :])
    s1_ref[...] = jnp.sum(tg * tra[0:1, :_D] + a1 * tra[0:1, _D:], axis=1,
                          keepdims=True)
    s2_ref[...] = jnp.sum(tg * tra[1:2, :_D] + a2 * tra[1:2, _D:], axis=1,
                          keepdims=True)


def _full(shape):
    return pl.BlockSpec(shape, lambda i: tuple(0 for _ in shape))


def _tc1(p, r0, w):
    g = _N // _B
    eshape = jax.ShapeDtypeStruct((_N, _D), jnp.float32)
    return pl.pallas_call(
        _tc1_body,
        grid=(g,),
        in_specs=[
            pl.BlockSpec((3, _NC, _B, _D), lambda i: (0, 0, i, 0)),
            _full((3, _D)),
            _full((_D, _D)),
        ],
        out_specs=[pl.BlockSpec((_B, _D), lambda i: (i, 0))] * 3,
        out_shape=[eshape] * 3,
    )(p, r0, w)


def _tc2(p, r1, w, base, e10, e11, e12):
    g = _N // _B
    return pl.pallas_call(
        _tc2_body,
        grid=(g,),
        in_specs=[
            pl.BlockSpec((3, _NC, _B, _D), lambda i: (0, 0, i, 0)),
            _full((3, _D)),
            _full((_D, _D)),
            pl.BlockSpec((_B, _D), lambda i: (i, 0)),
            pl.BlockSpec((_B, _D), lambda i: (i, 0)),
            pl.BlockSpec((_B, _D), lambda i: (i, 0)),
            pl.BlockSpec((_B, _D), lambda i: (i, 0)),
        ],
        out_specs=pl.BlockSpec((_B, 3, _D), lambda i: (i, 0, 0)),
        out_shape=jax.ShapeDtypeStruct((_N, 3, _D), jnp.float32),
    )(p, r1, w, base, e10, e11, e12)


def _rela_chain(r0, w0, w1):
    return pl.pallas_call(
        _rela_body,
        out_shape=[jax.ShapeDtypeStruct((3, _D), jnp.float32)] * 2,
    )(r0, w0, w1)


def _gru_scores(final, g0, g1, g2, gb, tra):
    g = _NU // _B
    return pl.pallas_call(
        _gru_body,
        grid=(g,),
        in_specs=[
            pl.BlockSpec((_B, 3, _D), lambda i: (i, 0, 0)),
            _full((_D, _D)),
            _full((_D, _D)),
            _full((_D, _D)),
            _full((3, _D)),
            _full((2, 2 * _D)),
        ],
        out_specs=[pl.BlockSpec((_B, 1), lambda i: (i, 0))] * 2,
        out_shape=[jax.ShapeDtypeStruct((_NU, 1), jnp.float32)] * 2,
    )(final, g0, g1, g2, gb, tra)


def kernel(edge_index_0, edge_index_1, edge_index_2, edge_val_0, edge_val_1,
           edge_val_2, user_embedding, item_embedding, relation_embedding,
           W_gc_0, W_gc_1, W_rel_0, W_rel_1, gru_w0, gru_w1, gru_w2, gru_b,
           tra):
    base = jnp.concatenate([user_embedding, item_embedding], axis=0)
    d0, s0 = edge_index_0[0], edge_index_0[1]
    d1, s1 = edge_index_1[0], edge_index_1[1]
    d2, s2 = edge_index_2[0], edge_index_2[1]

    r1, rmean = _rela_chain(relation_embedding, W_rel_0, W_rel_1)

    p1 = _spmm3(base, base, base, s0, d0, edge_val_0, s1, d1, edge_val_1,
                s2, d2, edge_val_2)
    e10, e11, e12 = _tc1(p1, relation_embedding, W_gc_0)

    p2 = _spmm3(e10, e11, e12, s0, d0, edge_val_0, s1, d1, edge_val_1,
                s2, d2, edge_val_2)
    final = _tc2(p2, r1, W_gc_1, base, e10, e11, e12)

    sc1, sc2 = _gru_scores(final, gru_w0, gru_w1, gru_w2, gru_b, tra)

    u_g = final[:_NU]
    i_g = jnp.concatenate([final[_NU:], jnp.zeros((1, 3, _D), jnp.float32)],
                          axis=0)
    rela_out = rmean.reshape(3, 1, _D)
    return (u_g, i_g, rela_out, sc1.reshape(_NU), sc2.reshape(_NU))


# speedup vs baseline: 1.0265x; 1.0265x over previous
import jax
import jax.numpy as jnp
from jax import lax
from jax.experimental import pallas as pl
from jax.experimental.pallas import tpu as pltpu
from jax.experimental.pallas import tpu_sc as plsc

_N = 10000
_D = 128
_E = 320000
_NC = 2
_NS = 16
_CHUNK = 128
_EC = _E // _NC
_NCHUNK = _EC // _CHUNK
_KFULL = _NCHUNK // _NS
_REM = _NCHUNK - _KFULL * _NS
_SB = 624
_ZR = 16

_NU = 5000
_LANES = _D // 16


def _spmm_body(x0, x1, x2, s0, d0, v0, s1, d1, v1, s2, d2, v2, out,
               idx_s, idx_d, vbuf, rows, acc,
               g0, g1, g2, c0, c1, c2, p0, p1, p2):
    cid = lax.axis_index("c")
    sid = lax.axis_index("s")
    strip = sid * _SB
    tail_base = _NS * _SB
    gsem = (g0, g1, g2)
    csem = (c0, c1, c2)
    psem = (p0, p1, p2)

    z16 = jnp.zeros((16,), jnp.float32)
    xs = (x0, x1, x2)
    srcs = (s0, s1, s2)
    dsts = (d0, d1, d2)
    vals = (v0, v1, v2)

    for rel in range(3):
        xr, sr, dr, vr = xs[rel], srcs[rel], dsts[rel], vals[rel]

        for r in range(_ZR):
            for j in range(_LANES):
                rows[0, r, pl.ds(j * 16, 16)] = z16

        def zero_body(t, _):
            pltpu.async_copy(rows.at[0, pl.ds(0, _ZR)],
                             acc.at[pl.ds(strip + t * _ZR, _ZR)], g0)
            return 0
        lax.fori_loop(0, _SB // _ZR, zero_body, 0)

        @pl.when(sid == _NS - 1)
        def _():
            pltpu.async_copy(rows.at[0, pl.ds(0, _ZR)],
                             acc.at[pl.ds(tail_base, _ZR)], g0)

        def zero_wait(t, _):
            pltpu.make_async_copy(rows.at[0, pl.ds(0, _ZR)],
                                  acc.at[pl.ds(strip, _ZR)], g0).wait()
            return 0
        lax.fori_loop(0, _SB // _ZR, zero_wait, 0)

        @pl.when(sid == _NS - 1)
        def _():
            pltpu.make_async_copy(rows.at[0, pl.ds(0, _ZR)],
                                  acc.at[pl.ds(tail_base, _ZR)], g0).wait()
        plsc.subcore_barrier()

        def ebase(k):
            return cid * _EC + (k * _NS + sid) * _CHUNK

        def start_gather(b):
            pltpu.async_copy(xr.at[idx_s.at[b]], rows.at[b], gsem[b])

        def wait_gather(b):
            pltpu.make_async_copy(xr.at[idx_s.at[b]], rows.at[b],
                                  gsem[b]).wait()

        def start_pf(b, k):
            pltpu.async_copy(sr.at[pl.ds(ebase(k), _CHUNK)], idx_s.at[b],
                             psem[b])
            pltpu.async_copy(vr.at[pl.ds(ebase(k), _CHUNK)], vbuf.at[b],
                             psem[b])
            pltpu.async_copy(dr.at[pl.ds(ebase(k), _CHUNK)], idx_d.at[b],
                             psem[b])

        def wait_pf(b):
            pltpu.make_async_copy(sr.at[pl.ds(0, _CHUNK)], idx_s.at[b],
                                  psem[b]).wait()
            pltpu.make_async_copy(vr.at[pl.ds(0, _CHUNK)], vbuf.at[b],
                                  psem[b]).wait()
            pltpu.make_async_copy(dr.at[pl.ds(0, _CHUNK)], idx_d.at[b],
                                  psem[b]).wait()

        def mult(b):
            @plsc.parallel_loop(0, _CHUNK, 1, unroll=4)
            def _mul(e):
                vb = plsc.load_gather(vbuf.at[b],
                                      [jnp.full((16,), e, jnp.int32)])
                for j in range(_LANES):
                    sl = pl.ds(j * 16, 16)
                    rows[b, e, sl] = rows[b, e, sl] * vb

        def start_scatter(b):
            pltpu.async_copy(rows.at[b], acc.at[idx_d.at[b]], csem[b],
                             add=True)

        def wait_scatter(b):
            pltpu.make_async_copy(xr.at[pl.ds(0, _CHUNK)], rows.at[b],
                                  csem[b]).wait()

        start_pf(0, 0)
        wait_pf(0)
        start_gather(0)
        start_pf(1, 1)
        wait_gather(0)
        wait_pf(1)
        start_gather(1)
        start_pf(2, 2)
        mult(0)
        start_scatter(0)
        wait_gather(1)
        wait_pf(2)
        start_gather(2)
        mult(1)
        start_scatter(1)

        def tri_body(jj, _):
            for i in (0, 1, 2):
                k = 3 * jj + 2 + i
                b = (2 + i) % 3
                o = i
                wait_scatter(o)
                start_pf(o, k + 1)
                wait_gather(b)
                wait_pf(o)
                start_gather(o)
                mult(b)
                start_scatter(b)
            return 0
        lax.fori_loop(0, (_KFULL - 3) // 3, tri_body, 0)
        _lb = (_KFULL - 1) % 3
        wait_gather(_lb)
        mult(_lb)
        start_scatter(_lb)
        wait_scatter((_KFULL - 3) % 3)
        wait_scatter((_KFULL - 2) % 3)
        wait_scatter(_lb)

        if _REM:
            @pl.when(sid < _REM)
            def _():
                base = cid * _EC + (_KFULL * _NS + sid) * _CHUNK
                pltpu.sync_copy(sr.at[pl.ds(base, _CHUNK)], idx_s.at[0])
                pltpu.sync_copy(dr.at[pl.ds(base, _CHUNK)], idx_d.at[0])
                pltpu.sync_copy(vr.at[pl.ds(base, _CHUNK)], vbuf.at[0])
                pltpu.async_copy(xr.at[idx_s.at[0]], rows.at[0], g0).wait()
                mult(0)
                pltpu.sync_copy(rows.at[0], acc.at[idx_d.at[0]], add=True)

        plsc.subcore_barrier()
        pltpu.sync_copy(acc.at[pl.ds(strip, _SB)],
                        out.at[rel, cid, pl.ds(strip, _SB)])

        @pl.when(sid == _NS - 1)
        def _():
            pltpu.sync_copy(acc.at[pl.ds(tail_base, _N - _NS * _SB)],
                            out.at[rel, cid, pl.ds(tail_base,
                                                   _N - _NS * _SB)])
        plsc.subcore_barrier()


_spmm3_cache = []


def _spmm3(*args):
    if not _spmm3_cache:
        _spmm3_cache.append(pl.kernel(
            _spmm_body,
            out_type=jax.ShapeDtypeStruct((3, _NC, _N, _D), jnp.float32),
            mesh=plsc.VectorSubcoreMesh(core_axis_name="c",
                                        subcore_axis_name="s",
                                        num_cores=_NC, num_subcores=_NS),
            compiler_params=pltpu.CompilerParams(needs_layout_passes=False),
            scratch_types=[
                pltpu.VMEM((3, _CHUNK), jnp.int32),
                pltpu.VMEM((3, _CHUNK), jnp.int32),
                pltpu.VMEM((3, _CHUNK), jnp.float32),
                pltpu.VMEM((3, _CHUNK, _D), jnp.float32),
                pltpu.VMEM_SHARED((_N, _D), jnp.float32),
            ] + [pltpu.SemaphoreType.DMA] * 9,
        ))
    return _spmm3_cache[0](*args)


_B = 1000


def _gc_es(p_ref, r, w_ref):
    es = []
    w = w_ref[...]
    for i in range(3):
        e = p_ref[i, 0] + p_ref[i, 1]
        e = e * r[i:i + 1, :]
        e = jnp.dot(e, w, preferred_element_type=jnp.float32)
        e = jnp.where(e >= 0.0, e, 0.01 * e)
        es.append(e)
    return es


def _attend(es, scale):
    s = {}
    for a in range(3):
        for b in range(a, 3):
            s[(a, b)] = jnp.sum(es[a] * es[b], axis=1, keepdims=True) * scale
            s[(b, a)] = s[(a, b)]
    outs = []
    for a in range(3):
        m = jnp.maximum(jnp.maximum(s[(a, 0)], s[(a, 1)]), s[(a, 2)])
        w = [jnp.exp(s[(a, b)] - m) for b in range(3)]
        den = w[0] + w[1] + w[2]
        outs.append((w[0] * es[0] + w[1] * es[1] + w[2] * es[2]) / den)
    return outs


def _tc1_body(p_ref, r_ref, w_ref, o0, o1, o2):
    es = _gc_es(p_ref, r_ref[...], w_ref)
    egos = _attend(es, 128.0 ** -0.5)
    o0[...] = egos[0]
    o1[...] = egos[1]
    o2[...] = egos[2]


def _tc2_body(p_ref, r0_ref, w0_ref, w1_ref, w_ref, base_ref, e10, e11, e12,
              g0, g1, g2, gb_ref, tra_ref, out_ref, rm_out, s1_ref, s2_ref):
    r0 = r0_ref[...]
    r1 = jnp.dot(r0, w0_ref[...], preferred_element_type=jnp.float32)
    r2 = jnp.dot(r1, w1_ref[...], preferred_element_type=jnp.float32)
    rm_out[...] = (r0 + r1 + r2) / 3.0
    es = _gc_es(p_ref, r1, w_ref)
    egos = _attend(es, 128.0 ** -0.5)
    b = base_ref[...]
    alls = [b + (e10, e11, e12)[i][...] + egos[i] for i in range(3)]
    t = [jnp.sum(alls[2] * alls[j], axis=1, keepdims=True) for j in range(3)]
    m = jnp.maximum(jnp.maximum(t[0], t[1]), t[2])
    w = [jnp.exp(t[j] - m) for j in range(3)]
    den = w[0] + w[1] + w[2]
    mid2 = (w[0] * alls[0] + w[1] * alls[1] + w[2] * alls[2]) / den
    u0 = alls[0] / 3.0
    u1 = alls[1] / 3.0
    u2 = mid2 / 3.0
    out_ref[:, 0, :] = u0
    out_ref[:, 1, :] = u1
    out_ref[:, 2, :] = u2
    gb = gb_ref[...]
    tra = tra_ref[...]
    a1 = u0 * (jnp.dot(u0, g0[...], preferred_element_type=jnp.float32) + gb[0:1, :])
    a2 = u1 * (jnp.dot(u1, g1[...], preferred_element_type=jnp.float32) + gb[1:2, :])
    tg = u2 * (jnp.dot(u2, g2[...], preferred_element_type=jnp.float32) + gb[2:3, :])
    s1_ref[...] = jnp.sum(tg * tra[0:1, :_D] + a1 * tra[0:1, _D:], axis=1,
                          keepdims=True)
    s2_ref[...] = jnp.sum(tg * tra[1:2, :_D] + a2 * tra[1:2, _D:], axis=1,
                          keepdims=True)


def _full(shape):
    return pl.BlockSpec(shape, lambda i: tuple(0 for _ in shape))


def _tc1(p, r0, w):
    g = _N // _B
    eshape = jax.ShapeDtypeStruct((_N, _D), jnp.float32)
    return pl.pallas_call(
        _tc1_body,
        grid=(g,),
        in_specs=[
            pl.BlockSpec((3, _NC, _B, _D), lambda i: (0, 0, i, 0)),
            _full((3, _D)),
            _full((_D, _D)),
        ],
        out_specs=[pl.BlockSpec((_B, _D), lambda i: (i, 0))] * 3,
        out_shape=[eshape] * 3,
    )(p, r0, w)


def _tc2(p, r0, wr0, wr1, w, base, e10, e11, e12, g0, g1, g2, gb, tra):
    g = _N // _B
    return pl.pallas_call(
        _tc2_body,
        grid=(g,),
        in_specs=[
            pl.BlockSpec((3, _NC, _B, _D), lambda i: (0, 0, i, 0)),
            _full((3, _D)),
            _full((_D, _D)),
            _full((_D, _D)),
            _full((_D, _D)),
            pl.BlockSpec((_B, _D), lambda i: (i, 0)),
            pl.BlockSpec((_B, _D), lambda i: (i, 0)),
            pl.BlockSpec((_B, _D), lambda i: (i, 0)),
            pl.BlockSpec((_B, _D), lambda i: (i, 0)),
            _full((_D, _D)),
            _full((_D, _D)),
            _full((_D, _D)),
            _full((3, _D)),
            _full((2, 2 * _D)),
        ],
        out_specs=[
            pl.BlockSpec((_B, 3, _D), lambda i: (i, 0, 0)),
            pl.BlockSpec((3, _D), lambda i: (0, 0)),
            pl.BlockSpec((_B, 1), lambda i: (i, 0)),
            pl.BlockSpec((_B, 1), lambda i: (i, 0)),
        ],
        out_shape=[
            jax.ShapeDtypeStruct((_N, 3, _D), jnp.float32),
            jax.ShapeDtypeStruct((3, _D), jnp.float32),
            jax.ShapeDtypeStruct((_N, 1), jnp.float32),
            jax.ShapeDtypeStruct((_N, 1), jnp.float32),
        ],
    )(p, r0, wr0, wr1, w, base, e10, e11, e12, g0, g1, g2, gb, tra)


def kernel(edge_index_0, edge_index_1, edge_index_2, edge_val_0, edge_val_1,
           edge_val_2, user_embedding, item_embedding, relation_embedding,
           W_gc_0, W_gc_1, W_rel_0, W_rel_1, gru_w0, gru_w1, gru_w2, gru_b,
           tra):
    base = jnp.concatenate([user_embedding, item_embedding], axis=0)
    d0, s0 = edge_index_0[0], edge_index_0[1]
    d1, s1 = edge_index_1[0], edge_index_1[1]
    d2, s2 = edge_index_2[0], edge_index_2[1]

    p1 = _spmm3(base, base, base, s0, d0, edge_val_0, s1, d1, edge_val_1,
                s2, d2, edge_val_2)
    e10, e11, e12 = _tc1(p1, relation_embedding, W_gc_0)

    p2 = _spmm3(e10, e11, e12, s0, d0, edge_val_0, s1, d1, edge_val_1,
                s2, d2, edge_val_2)
    final, rmean, s1f, s2f = _tc2(p2, relation_embedding, W_rel_0, W_rel_1,
                                  W_gc_1, base, e10, e11, e12,
                                  gru_w0, gru_w1, gru_w2, gru_b, tra)
    sc1 = s1f[:_NU]
    sc2 = s2f[:_NU]

    u_g = final[:_NU]
    i_g = jnp.concatenate([final[_NU:], jnp.zeros((1, 3, _D), jnp.float32)],
                          axis=0)
    rela_out = rmean.reshape(3, 1, _D)
    return (u_g, i_g, rela_out, sc1.reshape(_NU), sc2.reshape(_NU))
